# baseline (speedup 1.0000x reference)
import functools

import jax
import jax.numpy as jnp
from jax import lax
from jax.experimental import pallas as pl
from jax.experimental.pallas import tpu as pltpu
from jax.experimental.pallas import tpu_sc as plsc


def _pos_encoding(context_size, embedding_size):
    positions = jnp.arange(context_size, dtype=jnp.float32)
    indices = jnp.arange(embedding_size // 2, dtype=jnp.float32)
    scaling_factor = 10000 ** (2 * indices / embedding_size)
    angles = positions[:, None] / scaling_factor
    pe = jnp.zeros((context_size, embedding_size), dtype=jnp.float32)
    pe = pe.at[:, 0::2].set(jnp.sin(angles))
    pe = pe.at[:, 1::2].set(jnp.cos(angles))
    return pe


@functools.cache
def _build(batch, ctx, vocab, emb):
    info = plsc.get_sparse_core_info()
    nc, ns, lanes = info.num_cores, info.num_subcores, info.num_lanes
    nw = nc * ns
    chunk = 128
    bg_n = batch // chunk
    eg_n = emb // 8
    groups = emb // lanes
    chunks_total = ctx * bg_n
    assert chunks_total % nw == 0
    n_chunks = chunks_total // nw
    rows_per_w = n_chunks * chunk

    mesh = plsc.VectorSubcoreMesh(core_axis_name="c", subcore_axis_name="s")

    @functools.partial(
        pl.kernel,
        out_type=jax.ShapeDtypeStruct(
            (ctx, eg_n, bg_n, 8, chunk), jnp.float32
        ),
        mesh=mesh,
        scratch_types=[
            pltpu.VMEM((ctx * emb,), jnp.float32),
            pltpu.VMEM((rows_per_w,), jnp.int32),
            pltpu.VMEM((chunk, emb), jnp.float32),
            pltpu.VMEM((chunk, emb), jnp.float32),
            pltpu.VMEM((eg_n, 8, chunk), jnp.float32),
            pltpu.VMEM((eg_n, 8, chunk), jnp.float32),
            pltpu.SemaphoreType.DMA,
            pltpu.SemaphoreType.DMA,
            pltpu.SemaphoreType.DMA,
            pltpu.SemaphoreType.DMA,
        ],
        compiler_params=pltpu.CompilerParams(
            use_tc_tiling_on_sc=False, needs_layout_passes=False
        ),
    )
    def k(table_hbm, idx_hbm, pe_hbm, out_hbm,
          pe_v, idx_v, rows0, rows1, tiles0, tiles1, sg0, sg1, ss0, ss1):
        rows = (rows0, rows1)
        tiles = (tiles0, tiles1)
        sg = (sg0, sg1)
        ss = (ss0, ss1)
        wid = lax.axis_index("s") * nc + lax.axis_index("c")
        pltpu.sync_copy(pe_hbm, pe_v)
        pltpu.sync_copy(idx_hbm.at[pl.ds(wid * rows_per_w, rows_per_w)], idx_v)

        def gather(c, b):
            pltpu.async_copy(
                table_hbm.at[idx_v.at[pl.ds(c * chunk, chunk)]], rows[b], sg[b]
            )

        gather(0, 0)
        gather(1, 1)
        iota = lax.iota(jnp.int32, lanes)
        zero = jnp.bitwise_and(iota, 0)
        egv = []
        esv = []
        for g in range(groups):
            e_vec = iota + g * lanes
            egv.append(lax.shift_right_logical(e_vec, 3))
            esv.append(jnp.bitwise_and(e_vec, 7))

        def pair_body(p, carry):
            for b in range(2):
                c = 2 * p + b
                cid = wid * n_chunks + c
                t = cid // bg_n
                bg = cid % bg_n
                pltpu.make_async_copy(
                    table_hbm.at[idx_v.at[pl.ds(c * chunk, chunk)]],
                    rows[b],
                    sg[b],
                ).wait()
                @pl.when(c >= 2)
                def _():
                    pltpu.make_async_copy(
                        tiles[b], out_hbm.at[t, :, bg], ss[b]
                    ).wait()

                pev = [
                    pe_v[pl.ds(t * emb + g * lanes, lanes)]
                    for g in range(groups)
                ]

                @plsc.parallel_loop(0, chunk // 8, unroll=2)
                def bl_body(bl8):
                    base = bl8 * 8
                    for r in range(8):
                        blv = zero + (base + r)
                        for g in range(groups):
                            v = rows[b][base + r, pl.ds(g * lanes, lanes)]
                            plsc.store_scatter(
                                tiles[b], [egv[g], esv[g], blv], v + pev[g]
                            )

                nxt = c + 2

                @pl.when(nxt < n_chunks)
                def _():
                    gather(nxt, b)

                pltpu.async_copy(tiles[b], out_hbm.at[t, :, bg], ss[b])
            return carry

        lax.fori_loop(0, n_chunks // 2, pair_body, 0)
        for b in range(2):
            c = n_chunks - 2 + b
            cid = wid * n_chunks + c
            pltpu.make_async_copy(
                tiles[b], out_hbm.at[cid // bg_n, :, cid % bg_n], ss[b]
            ).wait()

    return k


def kernel(x, table):
    batch, ctx = x.shape
    vocab, emb = table.shape
    pe = _pos_encoding(ctx, emb).reshape(-1)
    idx = x.T.reshape(-1)
    out5 = _build(batch, ctx, vocab, emb)(table, idx, pe)
    return out5.transpose(2, 4, 0, 1, 3).reshape(batch, ctx, emb)


# speedup vs baseline: 1.0393x; 1.0393x over previous
import functools

import jax
import jax.numpy as jnp
from jax import lax
from jax.experimental import pallas as pl
from jax.experimental.pallas import tpu as pltpu
from jax.experimental.pallas import tpu_sc as plsc


def _pos_encoding(context_size, embedding_size):
    positions = jnp.arange(context_size, dtype=jnp.float32)
    indices = jnp.arange(embedding_size // 2, dtype=jnp.float32)
    scaling_factor = 10000 ** (2 * indices / embedding_size)
    angles = positions[:, None] / scaling_factor
    pe = jnp.zeros((context_size, embedding_size), dtype=jnp.float32)
    pe = pe.at[:, 0::2].set(jnp.sin(angles))
    pe = pe.at[:, 1::2].set(jnp.cos(angles))
    return pe


@functools.cache
def _build(batch, ctx, vocab, emb):
    info = plsc.get_sparse_core_info()
    nc, ns, lanes = info.num_cores, info.num_subcores, info.num_lanes
    nw = nc * ns
    chunk = 128
    bg_n = batch // chunk
    eg_n = emb // 8
    groups = emb // lanes
    chunks_total = ctx * bg_n
    assert chunks_total % nw == 0
    n_chunks = chunks_total // nw
    rows_per_w = n_chunks * chunk

    mesh = plsc.VectorSubcoreMesh(core_axis_name="c", subcore_axis_name="s")

    @functools.partial(
        pl.kernel,
        out_type=jax.ShapeDtypeStruct(
            (ctx, eg_n, bg_n, 8, chunk), jnp.float32
        ),
        mesh=mesh,
        scratch_types=[
            pltpu.VMEM((ctx * emb,), jnp.float32),
            pltpu.VMEM((rows_per_w,), jnp.int32),
            pltpu.VMEM((chunk, emb), jnp.float32),
            pltpu.VMEM((chunk, emb), jnp.float32),
            pltpu.VMEM((eg_n, 8, chunk), jnp.float32),
            pltpu.VMEM((eg_n, 8, chunk), jnp.float32),
            pltpu.SemaphoreType.DMA,
            pltpu.SemaphoreType.DMA,
            pltpu.SemaphoreType.DMA,
            pltpu.SemaphoreType.DMA,
        ],
        compiler_params=pltpu.CompilerParams(
            use_tc_tiling_on_sc=False, needs_layout_passes=False
        ),
    )
    def k(table_hbm, idx_hbm, pe_hbm, out_hbm,
          pe_v, idx_v, rows0, rows1, tiles0, tiles1, sg0, sg1, ss0, ss1):
        rows = (rows0, rows1)
        tiles = (tiles0, tiles1)
        sg = (sg0, sg1)
        ss = (ss0, ss1)
        wid = lax.axis_index("s") * nc + lax.axis_index("c")
        pltpu.sync_copy(pe_hbm, pe_v)
        pltpu.sync_copy(idx_hbm.at[pl.ds(wid * rows_per_w, rows_per_w)], idx_v)

        def gather(c, b):
            pltpu.async_copy(
                table_hbm.at[idx_v.at[pl.ds(c * chunk, chunk)]], rows[b], sg[b]
            )

        gather(0, 0)
        gather(1, 1)
        iota = lax.iota(jnp.int32, lanes)
        zero = jnp.bitwise_and(iota, 0)
        rsplat = [zero + r for r in range(8)]
        egv = []
        esv = []
        for g in range(groups):
            e_vec = iota + g * lanes
            egv.append(lax.shift_right_logical(e_vec, 3))
            esv.append(jnp.bitwise_and(e_vec, 7))

        def pair_body(p, carry):
            for b in range(2):
                c = 2 * p + b
                cid = wid * n_chunks + c
                t = cid // bg_n
                bg = cid % bg_n
                pltpu.make_async_copy(
                    table_hbm.at[idx_v.at[pl.ds(c * chunk, chunk)]],
                    rows[b],
                    sg[b],
                ).wait()
                @pl.when(c >= 2)
                def _():
                    pltpu.make_async_copy(
                        tiles[b], out_hbm.at[t, :, bg], ss[b]
                    ).wait()

                pev = [
                    pe_v[pl.ds(t * emb + g * lanes, lanes)]
                    for g in range(groups)
                ]

                @plsc.parallel_loop(0, chunk // 8, unroll=2)
                def bl_body(bl8):
                    base = bl8 * 8
                    for r in range(8):
                        for g in range(groups):
                            v = rows[b][base + r, pl.ds(g * lanes, lanes)]
                            plsc.store_scatter(
                                tiles[b],
                                [egv[g], esv[g], rsplat[r] + base],
                                v + pev[g],
                            )

                nxt = c + 2

                @pl.when(nxt < n_chunks)
                def _():
                    gather(nxt, b)

                pltpu.async_copy(tiles[b], out_hbm.at[t, :, bg], ss[b])
            return carry

        lax.fori_loop(0, n_chunks // 2, pair_body, 0)
        for b in range(2):
            c = n_chunks - 2 + b
            cid = wid * n_chunks + c
            pltpu.make_async_copy(
                tiles[b], out_hbm.at[cid // bg_n, :, cid % bg_n], ss[b]
            ).wait()

    return k


def kernel(x, table):
    batch, ctx = x.shape
    vocab, emb = table.shape
    pe = _pos_encoding(ctx, emb).reshape(-1)
    idx = x.T.reshape(-1)
    out5 = _build(batch, ctx, vocab, emb)(table, idx, pe)
    return out5.transpose(2, 4, 0, 1, 3).reshape(batch, ctx, emb)
